# baseline (speedup 1.0000x reference)
import functools

import jax
import jax.numpy as jnp
from jax import lax
from jax.experimental import pallas as pl
from jax.experimental.pallas import tpu as pltpu
from jax.experimental.pallas import tpu_sc as plsc

HIDDEN = 1024
FFN = 4096
E = 8
TOPK = 2

BLK = 128
FBLK = 2048
RBLK = 512


def _routing_kernel(x_ref, gw_ref, rw_ref, ids_ref, rank_ref, cnt_ref, acc_ref):
    i = pl.program_id(0)
    x = x_ref[...]
    logits = lax.dot_general(x, gw_ref[...], (((1,), (1,)), ((), ())),
                             preferred_element_type=jnp.float32)
    m = jnp.max(logits, axis=-1, keepdims=True)
    p = jnp.exp(logits - m)
    p = p / jnp.sum(p, axis=-1, keepdims=True)
    v1 = p[:, 0:1]
    i1 = jnp.zeros_like(v1, dtype=jnp.int32)
    for e in range(1, E):
        better = p[:, e:e + 1] > v1
        v1 = jnp.where(better, p[:, e:e + 1], v1)
        i1 = jnp.where(better, e, i1)
    neg = jnp.float32(-1.0)
    v2 = jnp.where(i1 == 0, neg, p[:, 0:1])
    i2 = jnp.zeros_like(i1)
    for e in range(1, E):
        cand = jnp.where(i1 == e, neg, p[:, e:e + 1])
        better = cand > v2
        v2 = jnp.where(better, cand, v2)
        i2 = jnp.where(better, e, i2)
    s = v1 + v2
    rw_ref[...] = jnp.concatenate([v1 / s, v2 / s], axis=1)
    ids_ref[...] = jnp.concatenate([i1, i2], axis=1)
    cols = lax.broadcasted_iota(jnp.int32, (RBLK, E), 1)
    oh = ((cols == i1) | (cols == i2)).astype(jnp.float32)
    ri = lax.broadcasted_iota(jnp.int32, (RBLK, RBLK), 0)
    cj = lax.broadcasted_iota(jnp.int32, (RBLK, RBLK), 1)
    tril = (ri > cj).astype(jnp.float32)
    excl = lax.dot_general(tril, oh, (((1,), (0,)), ((), ())),
                           preferred_element_type=jnp.float32)

    @pl.when(i == 0)
    def _():
        acc_ref[...] = jnp.zeros_like(acc_ref)

    base = acc_ref[...]
    tot = excl + base
    rank0 = jnp.sum(jnp.where(cols == i1, tot, 0.0), axis=1, keepdims=True)
    rank1 = jnp.sum(jnp.where(cols == i2, tot, 0.0), axis=1, keepdims=True)
    rank_ref[...] = jnp.concatenate([rank0, rank1], axis=1).astype(jnp.int32)
    newbase = base + jnp.sum(oh, axis=0, keepdims=True)
    acc_ref[...] = newbase
    cnt_ref[...] = newbase.astype(jnp.int32)


def _routing(xt, gate_w):
    t = xt.shape[0]
    grid = (t // RBLK,)
    rw, ids, ranks, counts = pl.pallas_call(
        _routing_kernel,
        grid=grid,
        in_specs=[
            pl.BlockSpec((RBLK, HIDDEN), lambda i: (i, 0)),
            pl.BlockSpec((E, HIDDEN), lambda i: (0, 0)),
        ],
        out_specs=[
            pl.BlockSpec((RBLK, TOPK), lambda i: (i, 0)),
            pl.BlockSpec((RBLK, TOPK), lambda i: (i, 0)),
            pl.BlockSpec((RBLK, TOPK), lambda i: (i, 0)),
            pl.BlockSpec((1, E), lambda i: (0, 0)),
        ],
        out_shape=[
            jax.ShapeDtypeStruct((t, TOPK), jnp.float32),
            jax.ShapeDtypeStruct((t, TOPK), jnp.int32),
            jax.ShapeDtypeStruct((t, TOPK), jnp.int32),
            jax.ShapeDtypeStruct((1, E), jnp.int32),
        ],
        scratch_shapes=[pltpu.VMEM((1, E), jnp.float32)],
        compiler_params=pltpu.CompilerParams(
            dimension_semantics=("arbitrary",)),
    )(xt, gate_w)
    return rw, ids, ranks, counts


def _dispatch_plan(ids, ranks, counts, t):
    n = t * TOPK
    nb = n // BLK + E
    p_rows = nb * BLK
    flat_e = ids.reshape(-1)
    counts = counts.reshape(E)
    padded = ((counts + BLK - 1) // BLK) * BLK
    offs = jnp.concatenate([jnp.zeros((1,), jnp.int32),
                            jnp.cumsum(padded)[:-1].astype(jnp.int32)])
    pos = offs[flat_e] + ranks.reshape(-1)
    end_blk = jnp.cumsum(padded // BLK)
    blk_ids = jnp.arange(nb, dtype=jnp.int32)
    block_expert = jnp.sum(blk_ids[:, None] >= end_blk[None, :],
                           axis=1).astype(jnp.int32)
    block_expert = jnp.minimum(block_expert, E - 1)
    return pos.reshape(t, TOPK), block_expert, nb


def _ffn_a_kernel(be_ref, xs_ref, w1_ref, w3_ref, h_ref):
    del be_ref
    x = xs_ref[...].astype(jnp.bfloat16)
    w1 = w1_ref[0].astype(jnp.bfloat16)
    w3 = w3_ref[0].astype(jnp.bfloat16)
    a = lax.dot_general(x, w1, (((1,), (1,)), ((), ())),
                        preferred_element_type=jnp.float32)
    b = lax.dot_general(x, w3, (((1,), (1,)), ((), ())),
                        preferred_element_type=jnp.float32)
    h = (a * jax.nn.sigmoid(a)) * b
    h_ref[...] = h.astype(jnp.bfloat16)


def _ffn_b_kernel(be_ref, h_ref, w2_ref, y_ref):
    del be_ref
    h = h_ref[...]
    w2 = w2_ref[0].astype(jnp.bfloat16)
    y_ref[...] = lax.dot_general(h, w2, (((1,), (1,)), ((), ())),
                                 preferred_element_type=jnp.float32)


def _grouped_ffn(xs, W1, W2, W3, block_expert, nb):
    p_rows = xs.shape[0]
    nf = FFN // FBLK
    h = pl.pallas_call(
        _ffn_a_kernel,
        grid_spec=pltpu.PrefetchScalarGridSpec(
            num_scalar_prefetch=1,
            grid=(nf, nb),
            in_specs=[
                pl.BlockSpec((BLK, HIDDEN), lambda f, b, be: (b, 0)),
                pl.BlockSpec((1, FBLK, HIDDEN), lambda f, b, be: (be[b], f, 0)),
                pl.BlockSpec((1, FBLK, HIDDEN), lambda f, b, be: (be[b], f, 0)),
            ],
            out_specs=pl.BlockSpec((BLK, FBLK), lambda f, b, be: (b, f)),
        ),
        out_shape=jax.ShapeDtypeStruct((p_rows, FFN), jnp.bfloat16),
        compiler_params=pltpu.CompilerParams(
            dimension_semantics=("arbitrary", "arbitrary")),
    )(block_expert, xs, W1, W3)
    y = pl.pallas_call(
        _ffn_b_kernel,
        grid_spec=pltpu.PrefetchScalarGridSpec(
            num_scalar_prefetch=1,
            grid=(nb,),
            in_specs=[
                pl.BlockSpec((BLK, FFN), lambda b, be: (b, 0)),
                pl.BlockSpec((1, HIDDEN, FFN), lambda b, be: (be[b], 0, 0)),
            ],
            out_specs=pl.BlockSpec((BLK, HIDDEN), lambda b, be: (b, 0)),
        ),
        out_shape=jax.ShapeDtypeStruct((p_rows, HIDDEN), jnp.float32),
        compiler_params=pltpu.CompilerParams(
            dimension_semantics=("arbitrary",)),
    )(block_expert, h, W2)
    return y


def _sc_dispatch(xt, pos, p_rows):
    t = xt.shape[0]
    info = plsc.get_sparse_core_info()
    nc, ns = info.num_cores, info.num_subcores
    nw = nc * ns
    per_w = t // nw
    ch = 32
    mesh = plsc.VectorSubcoreMesh(core_axis_name="c", subcore_axis_name="s")

    @functools.partial(
        pl.kernel, mesh=mesh,
        out_type=jax.ShapeDtypeStruct((p_rows, HIDDEN), jnp.float32),
        scratch_types=[
            pltpu.VMEM((ch,), jnp.int32),
            pltpu.VMEM((ch,), jnp.int32),
            pltpu.VMEM((ch, HIDDEN), jnp.float32),
            pltpu.SemaphoreType.DMA,
            pltpu.SemaphoreType.DMA,
        ])
    def k(xt_hbm, pa_hbm, pb_hbm, xs_hbm, ia_v, ib_v, buf, sem_a, sem_b):
        wid = lax.axis_index("s") * nc + lax.axis_index("c")
        base0 = wid * per_w

        def chunk_body(ci, _):
            base = base0 + ci * ch
            pltpu.sync_copy(pa_hbm.at[pl.ds(base, ch)], ia_v)
            pltpu.sync_copy(pb_hbm.at[pl.ds(base, ch)], ib_v)
            pltpu.sync_copy(xt_hbm.at[pl.ds(base, ch)], buf)
            ca = pltpu.async_copy(buf, xs_hbm.at[ia_v], sem_a)
            cb = pltpu.async_copy(buf, xs_hbm.at[ib_v], sem_b)
            ca.wait()
            cb.wait()
            return 0

        lax.fori_loop(0, per_w // ch, chunk_body, 0)

    return k(xt, pos[:, 0] + 0, pos[:, 1] + 0)


def _sc_combine(y, pos, rw):
    t = pos.shape[0]
    info = plsc.get_sparse_core_info()
    nc, ns, nl = info.num_cores, info.num_subcores, info.num_lanes
    nw = nc * ns
    per_w = t // nw
    ch = 32
    mesh = plsc.VectorSubcoreMesh(core_axis_name="c", subcore_axis_name="s")

    @functools.partial(
        pl.kernel, mesh=mesh,
        out_type=jax.ShapeDtypeStruct((t, HIDDEN), jnp.float32),
        scratch_types=[
            pltpu.VMEM((ch,), jnp.int32),
            pltpu.VMEM((ch,), jnp.int32),
            pltpu.VMEM((ch, 16), jnp.float32),
            pltpu.VMEM((ch, 16), jnp.float32),
            pltpu.VMEM((ch, HIDDEN), jnp.float32),
            pltpu.VMEM((ch, HIDDEN), jnp.float32),
            pltpu.VMEM((ch, HIDDEN), jnp.float32),
            pltpu.SemaphoreType.DMA,
            pltpu.SemaphoreType.DMA,
        ])
    def k(y_hbm, pa_hbm, pb_hbm, wa_hbm, wb_hbm, out_hbm,
          ia_v, ib_v, wa_v, wb_v, bufa, bufb, bufo, sem_a, sem_b):
        wid = lax.axis_index("s") * nc + lax.axis_index("c")
        base0 = wid * per_w

        def chunk_body(ci, _):
            base = base0 + ci * ch
            pltpu.sync_copy(pa_hbm.at[pl.ds(base, ch)], ia_v)
            pltpu.sync_copy(pb_hbm.at[pl.ds(base, ch)], ib_v)
            pltpu.sync_copy(wa_hbm.at[pl.ds(base, ch)], wa_v)
            pltpu.sync_copy(wb_hbm.at[pl.ds(base, ch)], wb_v)
            ca = pltpu.async_copy(y_hbm.at[ia_v], bufa, sem_a)
            cb = pltpu.async_copy(y_hbm.at[ib_v], bufb, sem_b)
            ca.wait()
            cb.wait()

            def row_body(r, _):
                wva = wa_v[r]
                wvb = wb_v[r]

                def col_body(c, _):
                    for u in range(4):
                        o = (c * 4 + u) * nl
                        a = bufa[r, pl.ds(o, nl)]
                        b = bufb[r, pl.ds(o, nl)]
                        bufo[r, pl.ds(o, nl)] = a * wva + b * wvb
                    return 0

                lax.fori_loop(0, HIDDEN // (4 * nl), col_body, 0)
                return 0

            lax.fori_loop(0, ch, row_body, 0)
            pltpu.sync_copy(bufo, out_hbm.at[pl.ds(base, ch)])
            return 0

        lax.fori_loop(0, per_w // ch, chunk_body, 0)

    wexp = jnp.broadcast_to(rw[:, :, None], (t, TOPK, nl))
    return k(y, pos[:, 0] + 0, pos[:, 1] + 0,
             wexp[:, 0] + 0.0, wexp[:, 1] + 0.0)


def kernel(x, gate_w, W1, W2, W3):
    bs, sq, dim = x.shape
    t = bs * sq
    xt = x.reshape(t, dim)
    rw, ids, ranks, counts = _routing(xt, gate_w)
    pos, block_expert, nb = _dispatch_plan(ids, ranks, counts, t)
    xs = _sc_dispatch(xt, pos, nb * BLK)
    y = _grouped_ffn(xs, W1, W2, W3, block_expert, nb)
    final = _sc_combine(y, pos, rw)
    return final.reshape(bs, sq, dim), rw


# speedup vs baseline: 1.7092x; 1.7092x over previous
import functools

import jax
import jax.numpy as jnp
from jax import lax
from jax.experimental import pallas as pl
from jax.experimental.pallas import tpu as pltpu
from jax.experimental.pallas import tpu_sc as plsc

HIDDEN = 1024
FFN = 4096
E = 8
TOPK = 2

BLK = 256
FBLK = 2048
RBLK = 512


def _routing_kernel(x_ref, gw_ref, rw_ref, ids_ref, rank_ref, cnt_ref, acc_ref):
    i = pl.program_id(0)
    x = x_ref[...]
    logits = lax.dot_general(x, gw_ref[...], (((1,), (1,)), ((), ())),
                             preferred_element_type=jnp.float32)
    m = jnp.max(logits, axis=-1, keepdims=True)
    p = jnp.exp(logits - m)
    p = p / jnp.sum(p, axis=-1, keepdims=True)
    v1 = p[:, 0:1]
    i1 = jnp.zeros_like(v1, dtype=jnp.int32)
    for e in range(1, E):
        better = p[:, e:e + 1] > v1
        v1 = jnp.where(better, p[:, e:e + 1], v1)
        i1 = jnp.where(better, e, i1)
    neg = jnp.float32(-1.0)
    v2 = jnp.where(i1 == 0, neg, p[:, 0:1])
    i2 = jnp.zeros_like(i1)
    for e in range(1, E):
        cand = jnp.where(i1 == e, neg, p[:, e:e + 1])
        better = cand > v2
        v2 = jnp.where(better, cand, v2)
        i2 = jnp.where(better, e, i2)
    s = v1 + v2
    rw_ref[...] = jnp.concatenate([v1 / s, v2 / s], axis=1)
    ids_ref[...] = jnp.concatenate([i1, i2], axis=1)
    cols = lax.broadcasted_iota(jnp.int32, (RBLK, E), 1)
    oh = ((cols == i1) | (cols == i2)).astype(jnp.float32)
    ri = lax.broadcasted_iota(jnp.int32, (RBLK, RBLK), 0)
    cj = lax.broadcasted_iota(jnp.int32, (RBLK, RBLK), 1)
    tril = (ri > cj).astype(jnp.float32)
    excl = lax.dot_general(tril, oh, (((1,), (0,)), ((), ())),
                           preferred_element_type=jnp.float32)

    @pl.when(i == 0)
    def _():
        acc_ref[...] = jnp.zeros_like(acc_ref)

    base = acc_ref[...]
    tot = excl + base
    rank0 = jnp.sum(jnp.where(cols == i1, tot, 0.0), axis=1, keepdims=True)
    rank1 = jnp.sum(jnp.where(cols == i2, tot, 0.0), axis=1, keepdims=True)
    rank_ref[...] = jnp.concatenate([rank0, rank1], axis=1).astype(jnp.int32)
    newbase = base + jnp.sum(oh, axis=0, keepdims=True)
    acc_ref[...] = newbase
    cnt_ref[...] = newbase.astype(jnp.int32)


def _routing(xt, gate_w):
    t = xt.shape[0]
    grid = (t // RBLK,)
    rw, ids, ranks, counts = pl.pallas_call(
        _routing_kernel,
        grid=grid,
        in_specs=[
            pl.BlockSpec((RBLK, HIDDEN), lambda i: (i, 0)),
            pl.BlockSpec((E, HIDDEN), lambda i: (0, 0)),
        ],
        out_specs=[
            pl.BlockSpec((RBLK, TOPK), lambda i: (i, 0)),
            pl.BlockSpec((RBLK, TOPK), lambda i: (i, 0)),
            pl.BlockSpec((RBLK, TOPK), lambda i: (i, 0)),
            pl.BlockSpec((1, E), lambda i: (0, 0)),
        ],
        out_shape=[
            jax.ShapeDtypeStruct((t, TOPK), jnp.float32),
            jax.ShapeDtypeStruct((t, TOPK), jnp.int32),
            jax.ShapeDtypeStruct((t, TOPK), jnp.int32),
            jax.ShapeDtypeStruct((1, E), jnp.int32),
        ],
        scratch_shapes=[pltpu.VMEM((1, E), jnp.float32)],
        compiler_params=pltpu.CompilerParams(
            dimension_semantics=("arbitrary",)),
    )(xt, gate_w)
    return rw, ids, ranks, counts


def _dispatch_plan(ids, ranks, counts, t):
    n = t * TOPK
    nb = n // BLK + E
    p_rows = nb * BLK
    flat_e = ids.reshape(-1)
    counts = counts.reshape(E)
    padded = ((counts + BLK - 1) // BLK) * BLK
    offs = jnp.concatenate([jnp.zeros((1,), jnp.int32),
                            jnp.cumsum(padded)[:-1].astype(jnp.int32)])
    pos = offs[flat_e] + ranks.reshape(-1)
    end_blk = jnp.cumsum(padded // BLK)
    used = end_blk[-1]
    blk_ids = jnp.arange(nb, dtype=jnp.int32)
    block_expert = jnp.sum(blk_ids[:, None] >= end_blk[None, :],
                           axis=1).astype(jnp.int32)
    block_expert = jnp.minimum(block_expert, E - 1)
    be_last = block_expert[jnp.maximum(used - 1, 0)]
    block_expert = jnp.where(blk_ids < used, block_expert, be_last)
    block_valid = (blk_ids < used).astype(jnp.int32)
    return pos.reshape(t, TOPK), block_expert, block_valid, nb


def _ffn_a_kernel(be_ref, bv_ref, xs_ref, w1_ref, w3_ref, h_ref):
    del be_ref

    @pl.when(bv_ref[pl.program_id(1)] != 0)
    def _():
        x = xs_ref[...].astype(jnp.bfloat16)
        w1 = w1_ref[0].astype(jnp.bfloat16)
        w3 = w3_ref[0].astype(jnp.bfloat16)
        a = lax.dot_general(x, w1, (((1,), (1,)), ((), ())),
                            preferred_element_type=jnp.float32)
        b = lax.dot_general(x, w3, (((1,), (1,)), ((), ())),
                            preferred_element_type=jnp.float32)
        h = (a * jax.nn.sigmoid(a)) * b
        h_ref[...] = h.astype(jnp.bfloat16)


def _ffn_b_kernel(be_ref, bv_ref, h_ref, w2_ref, y_ref):
    del be_ref

    @pl.when(bv_ref[pl.program_id(0)] != 0)
    def _():
        h = h_ref[...]
        w2 = w2_ref[0].astype(jnp.bfloat16)
        y_ref[...] = lax.dot_general(h, w2, (((1,), (1,)), ((), ())),
                                     preferred_element_type=jnp.float32)


def _grouped_ffn(xs, W1, W2, W3, block_expert, block_valid, nb):
    p_rows = xs.shape[0]
    nf = FFN // FBLK
    h = pl.pallas_call(
        _ffn_a_kernel,
        grid_spec=pltpu.PrefetchScalarGridSpec(
            num_scalar_prefetch=2,
            grid=(nf, nb),
            in_specs=[
                pl.BlockSpec((BLK, HIDDEN), lambda f, b, be, bv: (b, 0)),
                pl.BlockSpec((1, FBLK, HIDDEN),
                             lambda f, b, be, bv: (be[b], f, 0)),
                pl.BlockSpec((1, FBLK, HIDDEN),
                             lambda f, b, be, bv: (be[b], f, 0)),
            ],
            out_specs=pl.BlockSpec((BLK, FBLK), lambda f, b, be, bv: (b, f)),
        ),
        out_shape=jax.ShapeDtypeStruct((p_rows, FFN), jnp.bfloat16),
        compiler_params=pltpu.CompilerParams(
            dimension_semantics=("arbitrary", "arbitrary")),
    )(block_expert, block_valid, xs, W1, W3)
    y = pl.pallas_call(
        _ffn_b_kernel,
        grid_spec=pltpu.PrefetchScalarGridSpec(
            num_scalar_prefetch=2,
            grid=(nb,),
            in_specs=[
                pl.BlockSpec((BLK, FFN), lambda b, be, bv: (b, 0)),
                pl.BlockSpec((1, HIDDEN, FFN), lambda b, be, bv: (be[b], 0, 0)),
            ],
            out_specs=pl.BlockSpec((BLK, HIDDEN), lambda b, be, bv: (b, 0)),
        ),
        out_shape=jax.ShapeDtypeStruct((p_rows, HIDDEN), jnp.float32),
        compiler_params=pltpu.CompilerParams(
            dimension_semantics=("arbitrary",)),
    )(block_expert, block_valid, h, W2)
    return y


def _sc_dispatch(xt, pos, p_rows):
    t = xt.shape[0]
    info = plsc.get_sparse_core_info()
    nc, ns = info.num_cores, info.num_subcores
    nw = nc * ns
    per_w = t // nw
    ch = 32
    mesh = plsc.VectorSubcoreMesh(core_axis_name="c", subcore_axis_name="s")

    @functools.partial(
        pl.kernel, mesh=mesh,
        out_type=jax.ShapeDtypeStruct((p_rows, HIDDEN), jnp.float32),
        scratch_types=[
            pltpu.VMEM((ch,), jnp.int32),
            pltpu.VMEM((ch,), jnp.int32),
            pltpu.VMEM((ch, HIDDEN), jnp.float32),
            pltpu.SemaphoreType.DMA,
            pltpu.SemaphoreType.DMA,
        ])
    def k(xt_hbm, pa_hbm, pb_hbm, xs_hbm, ia_v, ib_v, buf, sem_a, sem_b):
        wid = lax.axis_index("s") * nc + lax.axis_index("c")
        base0 = wid * per_w

        def chunk_body(ci, _):
            base = base0 + ci * ch
            pltpu.sync_copy(pa_hbm.at[pl.ds(base, ch)], ia_v)
            pltpu.sync_copy(pb_hbm.at[pl.ds(base, ch)], ib_v)
            pltpu.sync_copy(xt_hbm.at[pl.ds(base, ch)], buf)
            ca = pltpu.async_copy(buf, xs_hbm.at[ia_v], sem_a)
            cb = pltpu.async_copy(buf, xs_hbm.at[ib_v], sem_b)
            ca.wait()
            cb.wait()
            return 0

        lax.fori_loop(0, per_w // ch, chunk_body, 0)

    return k(xt, pos[:, 0] + 0, pos[:, 1] + 0)


def _sc_combine(y, pos, rw):
    t = pos.shape[0]
    info = plsc.get_sparse_core_info()
    nc, ns, nl = info.num_cores, info.num_subcores, info.num_lanes
    nw = nc * ns
    per_w = t // nw
    ch = 32
    mesh = plsc.VectorSubcoreMesh(core_axis_name="c", subcore_axis_name="s")

    @functools.partial(
        pl.kernel, mesh=mesh,
        out_type=jax.ShapeDtypeStruct((t, HIDDEN), jnp.float32),
        scratch_types=[
            pltpu.VMEM((ch,), jnp.int32),
            pltpu.VMEM((ch,), jnp.int32),
            pltpu.VMEM((ch, 16), jnp.float32),
            pltpu.VMEM((ch, 16), jnp.float32),
            pltpu.VMEM((ch, HIDDEN), jnp.float32),
            pltpu.VMEM((ch, HIDDEN), jnp.float32),
            pltpu.VMEM((ch, HIDDEN), jnp.float32),
            pltpu.SemaphoreType.DMA,
            pltpu.SemaphoreType.DMA,
        ])
    def k(y_hbm, pa_hbm, pb_hbm, wa_hbm, wb_hbm, out_hbm,
          ia_v, ib_v, wa_v, wb_v, bufa, bufb, bufo, sem_a, sem_b):
        wid = lax.axis_index("s") * nc + lax.axis_index("c")
        base0 = wid * per_w

        def chunk_body(ci, _):
            base = base0 + ci * ch
            pltpu.sync_copy(pa_hbm.at[pl.ds(base, ch)], ia_v)
            pltpu.sync_copy(pb_hbm.at[pl.ds(base, ch)], ib_v)
            pltpu.sync_copy(wa_hbm.at[pl.ds(base, ch)], wa_v)
            pltpu.sync_copy(wb_hbm.at[pl.ds(base, ch)], wb_v)
            ca = pltpu.async_copy(y_hbm.at[ia_v], bufa, sem_a)
            cb = pltpu.async_copy(y_hbm.at[ib_v], bufb, sem_b)
            ca.wait()
            cb.wait()

            def row_body(r, _):
                wva = wa_v[r]
                wvb = wb_v[r]

                def col_body(c, _):
                    for u in range(4):
                        o = (c * 4 + u) * nl
                        a = bufa[r, pl.ds(o, nl)]
                        b = bufb[r, pl.ds(o, nl)]
                        bufo[r, pl.ds(o, nl)] = a * wva + b * wvb
                    return 0

                lax.fori_loop(0, HIDDEN // (4 * nl), col_body, 0)
                return 0

            lax.fori_loop(0, ch, row_body, 0)
            pltpu.sync_copy(bufo, out_hbm.at[pl.ds(base, ch)])
            return 0

        lax.fori_loop(0, per_w // ch, chunk_body, 0)

    wexp = jnp.broadcast_to(rw[:, :, None], (t, TOPK, nl))
    return k(y, pos[:, 0] + 0, pos[:, 1] + 0,
             wexp[:, 0] + 0.0, wexp[:, 1] + 0.0)


def kernel(x, gate_w, W1, W2, W3):
    bs, sq, dim = x.shape
    t = bs * sq
    xt = x.reshape(t, dim)
    rw, ids, ranks, counts = _routing(xt, gate_w)
    pos, block_expert, block_valid, nb = _dispatch_plan(ids, ranks, counts, t)
    xs = _sc_dispatch(xt, pos, nb * BLK)
    y = _grouped_ffn(xs, W1, W2, W3, block_expert, block_valid, nb)
    final = _sc_combine(y, pos, rw)
    return final.reshape(bs, sq, dim), rw
